# baseline (speedup 1.0000x reference)
import functools

import jax
import jax.numpy as jnp
from jax import lax
from jax.experimental import pallas as pl
from jax.experimental.pallas import tpu as pltpu
from jax.experimental.pallas import tpu_sc as plsc

B = 16384
D = 32
NC = 2
NS = 16
NW = NC * NS
BPW = B // NW
CHUNK = 128
NCHUNK = BPW // CHUNK


def _dot_body(trip_hbm, table_hbm, out_hbm, trip_v, idx_v, lp_v, rp_v,
              acc_v, sem):
    wid = lax.axis_index("s") * NC + lax.axis_index("c")
    base = wid * BPW

    pltpu.sync_copy(trip_hbm.at[pl.ds(base * 3, BPW * 3)], trip_v)

    iota = lax.iota(jnp.int32, 16)
    iota3 = iota * 3

    for k in range(NCHUNK):
        for jj in range(CHUNK // 16):
            fbase = iota3 + (k * CHUNK + jj * 16) * 3
            idx_v[k, pl.ds(jj * 16, 16)] = plsc.load_gather(
                trip_v, [fbase])
            idx_v[NCHUNK + k, pl.ds(jj * 16, 16)] = plsc.load_gather(
                trip_v, [fbase + 2])

    zero = jnp.zeros((16,), jnp.float32)
    for j in range(BPW // 16):
        acc_v[pl.ds(j * 16, 16)] = zero

    def dbody(d, carry):
        plane = table_hbm.at[d]
        copies = []
        for k in range(NCHUNK):
            copies.append(pltpu.async_copy(
                plane.at[idx_v.at[k]], lp_v.at[pl.ds(k * CHUNK, CHUNK)],
                sem))
            copies.append(pltpu.async_copy(
                plane.at[idx_v.at[NCHUNK + k]],
                rp_v.at[pl.ds(k * CHUNK, CHUNK)], sem))
        for c in copies:
            c.wait()

        def jbody(j, c2):
            s = pl.ds(j * 16, 16)
            acc_v[s] = acc_v[s] + lp_v[s] * rp_v[s]
            return c2

        lax.fori_loop(0, BPW // 16, jbody, 0)
        return carry

    lax.fori_loop(0, D, dbody, 0)

    pltpu.sync_copy(acc_v, out_hbm.at[pl.ds(base, BPW)])


def kernel(triplets, node_emb, vars):
    del vars
    mesh = plsc.VectorSubcoreMesh(core_axis_name="c", subcore_axis_name="s")
    f = functools.partial(
        pl.kernel,
        out_type=jax.ShapeDtypeStruct((B,), jnp.float32),
        mesh=mesh,
        compiler_params=pltpu.CompilerParams(
            needs_layout_passes=False, use_tc_tiling_on_sc=False),
        scratch_types=[
            pltpu.VMEM((BPW * 3,), jnp.int32),
            pltpu.VMEM((2 * NCHUNK, CHUNK), jnp.int32),
            pltpu.VMEM((BPW,), jnp.float32),
            pltpu.VMEM((BPW,), jnp.float32),
            pltpu.VMEM((BPW,), jnp.float32),
            pltpu.SemaphoreType.DMA,
        ],
    )(_dot_body)
    return f(triplets.reshape(-1), node_emb.T)


# speedup vs baseline: 4.9195x; 4.9195x over previous
import functools

import jax
import jax.numpy as jnp
from jax import lax
from jax.experimental import pallas as pl
from jax.experimental.pallas import tpu as pltpu
from jax.experimental.pallas import tpu_sc as plsc

B = 16384
D = 32
QW = 128
NQ = 250000
NC = 2
NS = 16
NW = NC * NS
BPW = B // NW
HALF = BPW // 2
CHUNK = 128


def _dot_body(trip_hbm, table_hbm, out_hbm, trip_v, idx_v, off_v, lbuf_v,
              rbuf_v, acc_v, sem):
    wid = lax.axis_index("s") * NC + lax.axis_index("c")
    base = wid * BPW

    pltpu.sync_copy(trip_hbm.at[pl.ds(base * 3, BPW * 3)], trip_v)

    iota = lax.iota(jnp.int32, 16)
    iota3 = iota * 3

    for k in range(BPW // CHUNK):
        for jj in range(CHUNK // 16):
            fbase = iota3 + (k * CHUNK + jj * 16) * 3
            lid = plsc.load_gather(trip_v, [fbase])
            rid = plsc.load_gather(trip_v, [fbase + 2])
            s = pl.ds(jj * 16, 16)
            idx_v[k, s] = lax.shift_right_logical(lid, 2)
            idx_v[4 + k, s] = lax.shift_right_logical(rid, 2)
            t = pl.ds(k * CHUNK + jj * 16, 16)
            off_v[0, t] = lax.shift_left(jnp.bitwise_and(lid, 3), 5)
            off_v[1, t] = lax.shift_left(jnp.bitwise_and(rid, 3), 5)

    for h in range(2):
        copies = []
        for k in range(2):
            copies.append(pltpu.async_copy(
                table_hbm.at[idx_v.at[2 * h + k]],
                lbuf_v.at[pl.ds(k * CHUNK, CHUNK)], sem))
            copies.append(pltpu.async_copy(
                table_hbm.at[idx_v.at[4 + 2 * h + k]],
                rbuf_v.at[pl.ds(k * CHUNK, CHUNK)], sem))
        for c in copies:
            c.wait()

        def block(j, carry, h=h):
            t0 = h * HALF + j * 16
            rows = iota + j * 16
            loff = plsc.load_gather(off_v, [jnp.zeros((16,), jnp.int32),
                                            iota + t0])
            roff = plsc.load_gather(off_v, [jnp.ones((16,), jnp.int32),
                                            iota + t0])
            acc = jnp.zeros((16,), jnp.float32)
            for d in range(D):
                l = plsc.load_gather(lbuf_v, [rows, loff + d])
                r = plsc.load_gather(rbuf_v, [rows, roff + d])
                acc = acc + l * r
            acc_v[pl.ds(t0, 16)] = acc
            return carry

        lax.fori_loop(0, HALF // 16, block, 0)

    pltpu.sync_copy(acc_v, out_hbm.at[pl.ds(base, BPW)])


def kernel(triplets, node_emb, vars):
    del vars
    mesh = plsc.VectorSubcoreMesh(core_axis_name="c", subcore_axis_name="s")
    f = functools.partial(
        pl.kernel,
        out_type=jax.ShapeDtypeStruct((B,), jnp.float32),
        mesh=mesh,
        compiler_params=pltpu.CompilerParams(
            needs_layout_passes=False, use_tc_tiling_on_sc=False),
        scratch_types=[
            pltpu.VMEM((BPW * 3,), jnp.int32),
            pltpu.VMEM((8, CHUNK), jnp.int32),
            pltpu.VMEM((2, BPW), jnp.int32),
            pltpu.VMEM((HALF, QW), jnp.float32),
            pltpu.VMEM((HALF, QW), jnp.float32),
            pltpu.VMEM((BPW,), jnp.float32),
            pltpu.SemaphoreType.DMA,
        ],
    )(_dot_body)
    return f(triplets.reshape(-1), node_emb.reshape(NQ, QW))
